# baseline (speedup 1.0000x reference)
import functools

import jax
import jax.numpy as jnp
from jax import lax
from jax.experimental import pallas as pl
from jax.experimental.pallas import tpu as pltpu


def _body(pcl_hbm, lab_ref, w_ref, pclab_ref, pcp_ref, pcc_ref, imw_ref,
          im_ref, out_ref, pbuf, sems, *, n, c, p, bn, grid):
    i = pl.program_id(0)
    slot = lax.rem(i, 2)
    nxt = lax.rem(i + 1, 2)
    tail = n - (grid - 1) * bn

    def full_copy(b, s):
        return pltpu.make_async_copy(pcl_hbm.at[pl.ds(b * bn, bn)],
                                     pbuf.at[s], sems.at[s])

    def tail_copy(s):
        return pltpu.make_async_copy(
            pcl_hbm.at[pl.ds((grid - 1) * bn, tail)],
            pbuf.at[s].at[pl.ds(0, tail)], sems.at[s])

    @pl.when(i == 0)
    def _():
        full_copy(0, 0).start()

    @pl.when(i < grid - 2)
    def _():
        full_copy(i + 1, nxt).start()

    @pl.when(i == grid - 2)
    def _():
        tail_copy(nxt).start()

    @pl.when(i < grid - 1)
    def _():
        full_copy(i, slot).wait()

    @pl.when(i == grid - 1)
    def _():
        tail_copy(slot).wait()

    x = pbuf.at[slot][:, 0:1]
    z = jnp.log(x)
    valid = i * bn + lax.broadcasted_iota(jnp.int32, (bn,), 0) < n
    wm = jnp.where(valid & (lab_ref[...] == 0), w_ref[...], 0.0)
    bg_part = lax.dot_general(
        wm.reshape(1, bn), z,
        dimension_numbers=(((1,), (0,)), ((), ())),
        preferred_element_type=jnp.float32)

    im_r = im_ref[...].reshape(1, c)
    gate = (im_r[:, 0:1] != 0.0).astype(jnp.float32)

    @pl.when(i == 0)
    def _():
        pclab = pclab_ref[...].reshape(1, p)
        imnz = (im_r != 0.0).astype(jnp.float32)
        onehot = (lax.broadcasted_iota(jnp.int32, (c, p), 0)
                  == pclab).astype(jnp.float32)
        im_at_nz = lax.dot_general(
            imnz, onehot,
            dimension_numbers=(((1,), (0,)), ((), ())),
            preferred_element_type=jnp.float32)
        pcp = pcp_ref[...].reshape(1, p)
        fg_mask = ((im_at_nz > 0.5) & (pclab > 0)
                   & (pcc_ref[...].reshape(1, p) > 0.0))
        fg = jnp.sum(
            jnp.where(fg_mask,
                      imw_ref[...].reshape(1, p) * jnp.log(pcp), 0.0),
            keepdims=True)
        out_ref[...] = fg * jnp.float32(-1.0 / n)

    out_ref[...] += (gate * bg_part) * jnp.float32(-1.0 / n)


@functools.partial(jax.jit, static_argnames=("n", "c", "p", "bn"))
def _loss(pcl_prob, labels, w, pc_labels, pc_probs, pc_count, img_w,
          im_labels, *, n, c, p, bn):
    grid = -(-n // bn)
    full = lambda i: (0,)
    out = pl.pallas_call(
        functools.partial(_body, n=n, c=c, p=p, bn=bn, grid=grid),
        grid=(grid,),
        in_specs=[
            pl.BlockSpec(memory_space=pl.ANY),
            pl.BlockSpec((bn,), lambda i: (i,)),
            pl.BlockSpec((bn,), lambda i: (i,)),
            pl.BlockSpec((p,), full),
            pl.BlockSpec((p,), full),
            pl.BlockSpec((p,), full),
            pl.BlockSpec((p,), full),
            pl.BlockSpec((c,), full),
        ],
        out_specs=pl.BlockSpec((1, 1), lambda i: (0, 0)),
        out_shape=jax.ShapeDtypeStruct((1, 1), jnp.float32),
        scratch_shapes=[
            pltpu.VMEM((2, bn, c), jnp.float32),
            pltpu.SemaphoreType.DMA((2,)),
        ],
    )(pcl_prob, labels, w, pc_labels, pc_probs, pc_count, img_w, im_labels)
    return out[0, 0]


def kernel(pcl_prob, labels, cls_loss_weights, gt_assignment, pc_labels,
           pc_probs, pc_count, img_cls_loss_weights, im_labels_real):
    n, c = pcl_prob.shape
    p = pc_labels.shape[0]
    return _loss(pcl_prob, labels, cls_loss_weights, pc_labels, pc_probs,
                 pc_count, img_cls_loss_weights, im_labels_real,
                 n=n, c=c, p=p, bn=4096)


# speedup vs baseline: 8.4682x; 8.4682x over previous
import functools

import jax
import jax.numpy as jnp
from jax import lax
from jax.experimental import pallas as pl


def _body(pclT_ref, lab_ref, w_ref, pclab_ref, pcp_ref, pcc_ref, imw_ref,
          im_ref, out_ref, *, n, c, p):
    z = jnp.log(pclT_ref[0:1, :]).reshape(n)
    wm = jnp.where(lab_ref[...] == 0, w_ref[...], 0.0)
    bg = jnp.sum(wm * z, keepdims=True).reshape(1, 1)

    im_r = im_ref[...].reshape(1, c)
    gate = (im_r[:, 0:1] != 0.0).astype(jnp.float32)

    pclab = pclab_ref[...].reshape(1, p)
    imnz = (im_r != 0.0).astype(jnp.float32)
    onehot = (lax.broadcasted_iota(jnp.int32, (c, p), 0)
              == pclab).astype(jnp.float32)
    im_at_nz = lax.dot_general(
        imnz, onehot,
        dimension_numbers=(((1,), (0,)), ((), ())),
        preferred_element_type=jnp.float32)
    fg_mask = ((im_at_nz > 0.5) & (pclab > 0)
               & (pcc_ref[...].reshape(1, p) > 0.0))
    fg = jnp.sum(
        jnp.where(fg_mask,
                  imw_ref[...].reshape(1, p) * jnp.log(pcp_ref[...].reshape(1, p)),
                  0.0),
        keepdims=True)

    out_ref[...] = (gate * bg + fg) * jnp.float32(-1.0 / n)


@functools.partial(jax.jit, static_argnames=("n", "c", "p"))
def _loss(pclT, labels, w, pc_labels, pc_probs, pc_count, img_w,
          im_labels, *, n, c, p):
    full1 = lambda i: (0,)
    out = pl.pallas_call(
        functools.partial(_body, n=n, c=c, p=p),
        grid=(1,),
        in_specs=[
            pl.BlockSpec((8, n), lambda i: (0, 0)),
            pl.BlockSpec((n,), full1),
            pl.BlockSpec((n,), full1),
            pl.BlockSpec((p,), full1),
            pl.BlockSpec((p,), full1),
            pl.BlockSpec((p,), full1),
            pl.BlockSpec((p,), full1),
            pl.BlockSpec((c,), full1),
        ],
        out_specs=pl.BlockSpec((1, 1), lambda i: (0, 0)),
        out_shape=jax.ShapeDtypeStruct((1, 1), jnp.float32),
    )(pclT, labels, w, pc_labels, pc_probs, pc_count, img_w, im_labels)
    return out[0, 0]


def kernel(pcl_prob, labels, cls_loss_weights, gt_assignment, pc_labels,
           pc_probs, pc_count, img_cls_loss_weights, im_labels_real):
    n, c = pcl_prob.shape
    p = pc_labels.shape[0]
    return _loss(pcl_prob.T, labels, cls_loss_weights, pc_labels, pc_probs,
                 pc_count, img_cls_loss_weights, im_labels_real,
                 n=n, c=c, p=p)
